# baseline (speedup 1.0000x reference)
import functools

import jax
import jax.numpy as jnp
from jax import lax
from jax.experimental import pallas as pl
from jax.experimental.pallas import tpu as pltpu
from jax.experimental.pallas import tpu_sc as plsc

B = 16384
P = 128
K = 16
S = 1000000
NC = 2
NS = 16
NW = NC * NS
BPW = B // NW
IDX_CHUNK = 128
NCHUNK = BPW // IDX_CHUNK

CH = 2048
NFULL = S // CH
TAIL = S - NFULL * CH
NCHTOT = NFULL + 1
ITERS = -(-NCHTOT // NW)
RPS = 128 // K
TR = S // RPS
CHR = CH // RPS
TAILR = TAIL // RPS
WPR = 64


def _repack_body(tt_hbm, lin_hbm, buf0, buf1, ln0, ln1, tbuf, tlin,
                 sem, osem):
    wid = lax.axis_index("s") * NC + lax.axis_index("c")
    bufs = [buf0, buf1]
    lns = [ln0, ln1]
    lanes = lax.broadcasted_iota(jnp.int32, (K,), 0)

    def chunk_id(j):
        return wid + NW * j

    def fire_in(j):
        c = chunk_id(j)

        @pl.when(c < NFULL)
        def _():
            pltpu.async_copy(tt_hbm.at[:, pl.ds(c * CH, CH)],
                             bufs[j % 2], sem)

        @pl.when(c == NFULL)
        def _():
            pltpu.async_copy(tt_hbm.at[:, pl.ds(NFULL * CH, TAIL)],
                             tbuf, sem)

    def wait_in(j):
        c = chunk_id(j)

        @pl.when(c < NFULL)
        def _():
            pltpu.make_async_copy(tt_hbm.at[:, pl.ds(0, CH)],
                                  bufs[j % 2], sem).wait()

        @pl.when(c == NFULL)
        def _():
            pltpu.make_async_copy(tt_hbm.at[:, pl.ds(NFULL * CH, TAIL)],
                                  tbuf, sem).wait()

    pat = (lanes // RPS) * WPR + (lanes % RPS) * RPS

    def process(j):
        c = chunk_id(j)

        @pl.when(c < NFULL)
        def _():
            buf, ln = bufs[j % 2], lns[j % 2]

            @plsc.parallel_loop(0, CH // K, unroll=4)
            def _(b):
                for k2 in range(K // 2):
                    va = buf[2 * k2, pl.ds(b * K, K)]
                    vb = buf[2 * k2 + 1, pl.ds(b * K, K)]
                    w = plsc.bitcast(
                        plsc.pack(va, vb, format=plsc.PackFormat.INTERLEAVED),
                        jnp.int32)
                    offs = pat + (b * 128 + k2)
                    plsc.store_scatter(ln, [offs], w)
            pltpu.async_copy(ln, lin_hbm.at[pl.ds(c * CHR * WPR, CHR * WPR)], osem)

        @pl.when(c == NFULL)
        def _():
            def blk(b, carry):
                for k2 in range(K // 2):
                    va = tbuf[2 * k2, pl.ds(b * K, K)]
                    vb = tbuf[2 * k2 + 1, pl.ds(b * K, K)]
                    w = plsc.bitcast(
                        plsc.pack(va, vb, format=plsc.PackFormat.INTERLEAVED),
                        jnp.int32)
                    offs = pat + (b * 128 + k2)
                    plsc.store_scatter(tlin, [offs], w)
                return carry

            lax.fori_loop(0, TAIL // K, blk, 0)
            pltpu.async_copy(tlin, lin_hbm.at[pl.ds(NFULL * CHR * WPR, TAILR * WPR)], osem)

    def wait_out(j):
        c = chunk_id(j)

        @pl.when(c < NFULL)
        def _():
            pltpu.make_async_copy(lns[j % 2],
                                  lin_hbm.at[pl.ds(0, CHR * WPR)], osem).wait()

        @pl.when(c == NFULL)
        def _():
            pltpu.make_async_copy(tlin,
                                  lin_hbm.at[pl.ds(0, TAILR * WPR)], osem).wait()

    fire_in(0)
    for j in range(ITERS):
        if j + 1 < ITERS:
            fire_in(j + 1)
        wait_in(j)
        if j >= 2:
            wait_out(j - 2)
        process(j)
    for j in range(max(ITERS - 2, 0), ITERS):
        wait_out(j)


def _repack(tt):
    mesh = plsc.VectorSubcoreMesh(core_axis_name="c", subcore_axis_name="s")
    run = functools.partial(
        pl.kernel,
        mesh=mesh,
        compiler_params=pltpu.CompilerParams(
            needs_layout_passes=False, use_tc_tiling_on_sc=True),
        out_type=jax.ShapeDtypeStruct((TR * WPR,), jnp.int32),
        scratch_types=[
            pltpu.VMEM((K, CH), jnp.float32),
            pltpu.VMEM((K, CH), jnp.float32),
            pltpu.VMEM((CHR * WPR,), jnp.int32),
            pltpu.VMEM((CHR * WPR,), jnp.int32),
            pltpu.VMEM((K, TAIL), jnp.float32),
            pltpu.VMEM((TAILR * WPR,), jnp.int32),
            pltpu.SemaphoreType.DMA,
            pltpu.SemaphoreType.DMA,
        ],
    )(_repack_body)
    return run(tt)


def _sc_body(tre_hbm, sid_hbm, zt_hbm, p2_hbm, out_hbm,
             sid_v, idx_v, g_v, zt_v, p2_v, out_v, sem, zsem):
    wid = lax.axis_index("s") * NC + lax.axis_index("c")
    base = wid * BPW

    pltpu.sync_copy(sid_hbm.at[pl.ds(base, BPW)], sid_v)
    zcp = pltpu.async_copy(zt_hbm.at[:, pl.ds(base, BPW)], zt_v, zsem)
    pltpu.sync_copy(p2_hbm.at[pl.ds(base, BPW)], p2_v)

    def mk_idx(c, carry):
        for j in range(IDX_CHUNK // K):
            v = sid_v[pl.ds(c * IDX_CHUNK + j * K, K)]
            idx_v[c, pl.ds(j * K, K)] = v // RPS
        return carry

    lax.fori_loop(0, NCHUNK, mk_idx, 0)

    copies = [
        pltpu.async_copy(tre_hbm.at[idx_v.at[c]],
                         g_v.at[pl.ds(c * IDX_CHUNK, IDX_CHUNK)], sem)
        for c in range(NCHUNK)
    ]
    zcp.wait()
    for cp in copies:
        cp.wait()

    lanes = lax.broadcasted_iota(jnp.int32, (K,), 0)

    @plsc.parallel_loop(0, BPW // K, unroll=4)
    def _(g):
        svec = sid_v[pl.ds(g * K, K)]
        col0 = (svec % RPS) * RPS
        rows = lanes + g * K
        acc = p2_v[pl.ds(g * K, K)]
        for k2 in range(K // 2):
            w = plsc.load_gather(g_v, [rows, col0 + k2])
            ta = plsc.bitcast(jax.lax.shift_left(w, 16), jnp.float32)
            tb = plsc.bitcast(
                jax.lax.bitwise_and(w, jnp.int32(-65536)), jnp.float32)
            acc = acc + ta * zt_v[2 * k2, pl.ds(g * K, K)]
            acc = acc + tb * zt_v[2 * k2 + 1, pl.ds(g * K, K)]
        out_v[pl.ds(g * K, K)] = acc
    pltpu.sync_copy(out_v, out_hbm.at[pl.ds(base, BPW)])


def _sc_partial(tre, sid, zt, p2):
    mesh = plsc.VectorSubcoreMesh(core_axis_name="c", subcore_axis_name="s")
    run = functools.partial(
        pl.kernel,
        mesh=mesh,
        compiler_params=pltpu.CompilerParams(
            needs_layout_passes=False, use_tc_tiling_on_sc=False),
        out_type=jax.ShapeDtypeStruct((B,), jnp.float32),
        scratch_types=[
            pltpu.VMEM((BPW,), jnp.int32),
            pltpu.VMEM((NCHUNK, IDX_CHUNK), jnp.int32),
            pltpu.VMEM((BPW, WPR), jnp.int32),
            pltpu.VMEM((K, BPW), jnp.float32),
            pltpu.VMEM((BPW,), jnp.float32),
            pltpu.VMEM((BPW,), jnp.float32),
            pltpu.SemaphoreType.DMA,
            pltpu.SemaphoreType.DMA,
        ],
    )(_sc_body)
    return run(tre, sid, zt, p2)


def _mv_body(x_ref, w_ref, o_ref):
    o_ref[...] = jax.lax.dot_general(
        x_ref[...], w_ref[...], (((1,), (0,)), ((), ())),
        preferred_element_type=jnp.float32)


def _tc_matvec(x, w_col):
    blk = 2048
    return pl.pallas_call(
        _mv_body,
        grid=(B // blk,),
        in_specs=[
            pl.BlockSpec((blk, P), lambda i: (i, 0)),
            pl.BlockSpec((P, 1), lambda i: (0, 0)),
        ],
        out_specs=pl.BlockSpec((blk, 1), lambda i: (i, 0)),
        out_shape=jax.ShapeDtypeStruct((B, 1), jnp.float32),
    )(x, w_col)


def kernel(x, z, sid, W_pop, table):
    lin = _repack(table.T).reshape(TR, WPR)
    p2 = _tc_matvec(x, W_pop.reshape(P, 1))
    return _sc_partial(lin, sid, z.T, p2.reshape(B))


# speedup vs baseline: 1.0160x; 1.0160x over previous
import functools

import jax
import jax.numpy as jnp
from jax import lax
from jax.experimental import pallas as pl
from jax.experimental.pallas import tpu as pltpu
from jax.experimental.pallas import tpu_sc as plsc

B = 16384
P = 128
K = 16
S = 1000000
NC = 2
NS = 16
NW = NC * NS
BPW = B // NW
IDX_CHUNK = 128
NCHUNK = BPW // IDX_CHUNK

CH = 2304
NFULL = S // CH
TAIL = S - NFULL * CH
NCHTOT = NFULL + 1
ITERS = -(-NCHTOT // NW)
RPS = 128 // K
TR = S // RPS
CHR = CH // RPS
TAILR = TAIL // RPS
WPR = 64


def _repack_body(tt_hbm, lin_hbm, buf0, buf1, ln0, ln1, tbuf, tlin,
                 sem, osem):
    wid = lax.axis_index("s") * NC + lax.axis_index("c")
    bufs = [buf0, buf1]
    lns = [ln0, ln1]
    lanes = lax.broadcasted_iota(jnp.int32, (K,), 0)

    def chunk_id(j):
        return wid + NW * j

    def fire_in(j):
        c = chunk_id(j)

        @pl.when(c < NFULL)
        def _():
            pltpu.async_copy(tt_hbm.at[:, pl.ds(c * CH, CH)],
                             bufs[j % 2], sem)

        @pl.when(c == NFULL)
        def _():
            pltpu.async_copy(tt_hbm.at[:, pl.ds(NFULL * CH, TAIL)],
                             tbuf, sem)

    def wait_in(j):
        c = chunk_id(j)

        @pl.when(c < NFULL)
        def _():
            pltpu.make_async_copy(tt_hbm.at[:, pl.ds(0, CH)],
                                  bufs[j % 2], sem).wait()

        @pl.when(c == NFULL)
        def _():
            pltpu.make_async_copy(tt_hbm.at[:, pl.ds(NFULL * CH, TAIL)],
                                  tbuf, sem).wait()

    pat = (lanes // RPS) * WPR + (lanes % RPS) * RPS

    def process(j):
        c = chunk_id(j)

        @pl.when(c < NFULL)
        def _():
            buf, ln = bufs[j % 2], lns[j % 2]

            @plsc.parallel_loop(0, CH // K, unroll=4)
            def _(b):
                for k2 in range(K // 2):
                    va = buf[2 * k2, pl.ds(b * K, K)]
                    vb = buf[2 * k2 + 1, pl.ds(b * K, K)]
                    w = plsc.bitcast(
                        plsc.pack(va, vb, format=plsc.PackFormat.INTERLEAVED),
                        jnp.int32)
                    offs = pat + (b * 128 + k2)
                    plsc.store_scatter(ln, [offs], w)
            pltpu.async_copy(ln, lin_hbm.at[pl.ds(c * CHR * WPR, CHR * WPR)], osem)

        @pl.when(c == NFULL)
        def _():
            def blk(b, carry):
                for k2 in range(K // 2):
                    va = tbuf[2 * k2, pl.ds(b * K, K)]
                    vb = tbuf[2 * k2 + 1, pl.ds(b * K, K)]
                    w = plsc.bitcast(
                        plsc.pack(va, vb, format=plsc.PackFormat.INTERLEAVED),
                        jnp.int32)
                    offs = pat + (b * 128 + k2)
                    plsc.store_scatter(tlin, [offs], w)
                return carry

            lax.fori_loop(0, TAIL // K, blk, 0)
            pltpu.async_copy(tlin, lin_hbm.at[pl.ds(NFULL * CHR * WPR, TAILR * WPR)], osem)

    def wait_out(j):
        c = chunk_id(j)

        @pl.when(c < NFULL)
        def _():
            pltpu.make_async_copy(lns[j % 2],
                                  lin_hbm.at[pl.ds(0, CHR * WPR)], osem).wait()

        @pl.when(c == NFULL)
        def _():
            pltpu.make_async_copy(tlin,
                                  lin_hbm.at[pl.ds(0, TAILR * WPR)], osem).wait()

    fire_in(0)
    for j in range(ITERS):
        if j + 1 < ITERS:
            fire_in(j + 1)
        wait_in(j)
        if j >= 2:
            wait_out(j - 2)
        process(j)
    for j in range(max(ITERS - 2, 0), ITERS):
        wait_out(j)


def _repack(tt):
    mesh = plsc.VectorSubcoreMesh(core_axis_name="c", subcore_axis_name="s")
    run = functools.partial(
        pl.kernel,
        mesh=mesh,
        compiler_params=pltpu.CompilerParams(
            needs_layout_passes=False, use_tc_tiling_on_sc=True),
        out_type=jax.ShapeDtypeStruct((TR * WPR,), jnp.int32),
        scratch_types=[
            pltpu.VMEM((K, CH), jnp.float32),
            pltpu.VMEM((K, CH), jnp.float32),
            pltpu.VMEM((CHR * WPR,), jnp.int32),
            pltpu.VMEM((CHR * WPR,), jnp.int32),
            pltpu.VMEM((K, TAIL), jnp.float32),
            pltpu.VMEM((TAILR * WPR,), jnp.int32),
            pltpu.SemaphoreType.DMA,
            pltpu.SemaphoreType.DMA,
        ],
    )(_repack_body)
    return run(tt)


def _sc_body(tre_hbm, sid_hbm, zt_hbm, p2_hbm, out_hbm,
             sid_v, idx_v, g_v, zt_v, p2_v, out_v, sem, zsem):
    wid = lax.axis_index("s") * NC + lax.axis_index("c")
    base = wid * BPW

    pltpu.sync_copy(sid_hbm.at[pl.ds(base, BPW)], sid_v)
    zcp = pltpu.async_copy(zt_hbm.at[:, pl.ds(base, BPW)], zt_v, zsem)
    pltpu.sync_copy(p2_hbm.at[pl.ds(base, BPW)], p2_v)

    def mk_idx(c, carry):
        for j in range(IDX_CHUNK // K):
            v = sid_v[pl.ds(c * IDX_CHUNK + j * K, K)]
            idx_v[c, pl.ds(j * K, K)] = v // RPS
        return carry

    lax.fori_loop(0, NCHUNK, mk_idx, 0)

    copies = [
        pltpu.async_copy(tre_hbm.at[idx_v.at[c]],
                         g_v.at[pl.ds(c * IDX_CHUNK, IDX_CHUNK)], sem)
        for c in range(NCHUNK)
    ]
    zcp.wait()
    for cp in copies:
        cp.wait()

    lanes = lax.broadcasted_iota(jnp.int32, (K,), 0)

    @plsc.parallel_loop(0, BPW // K, unroll=4)
    def _(g):
        svec = sid_v[pl.ds(g * K, K)]
        col0 = (svec % RPS) * RPS
        rows = lanes + g * K
        acc = p2_v[pl.ds(g * K, K)]
        for k2 in range(K // 2):
            w = plsc.load_gather(g_v, [rows, col0 + k2])
            ta = plsc.bitcast(jax.lax.shift_left(w, 16), jnp.float32)
            tb = plsc.bitcast(
                jax.lax.bitwise_and(w, jnp.int32(-65536)), jnp.float32)
            acc = acc + ta * zt_v[2 * k2, pl.ds(g * K, K)]
            acc = acc + tb * zt_v[2 * k2 + 1, pl.ds(g * K, K)]
        out_v[pl.ds(g * K, K)] = acc
    pltpu.sync_copy(out_v, out_hbm.at[pl.ds(base, BPW)])


def _sc_partial(tre, sid, zt, p2):
    mesh = plsc.VectorSubcoreMesh(core_axis_name="c", subcore_axis_name="s")
    run = functools.partial(
        pl.kernel,
        mesh=mesh,
        compiler_params=pltpu.CompilerParams(
            needs_layout_passes=False, use_tc_tiling_on_sc=False),
        out_type=jax.ShapeDtypeStruct((B,), jnp.float32),
        scratch_types=[
            pltpu.VMEM((BPW,), jnp.int32),
            pltpu.VMEM((NCHUNK, IDX_CHUNK), jnp.int32),
            pltpu.VMEM((BPW, WPR), jnp.int32),
            pltpu.VMEM((K, BPW), jnp.float32),
            pltpu.VMEM((BPW,), jnp.float32),
            pltpu.VMEM((BPW,), jnp.float32),
            pltpu.SemaphoreType.DMA,
            pltpu.SemaphoreType.DMA,
        ],
    )(_sc_body)
    return run(tre, sid, zt, p2)


def _mv_body(x_ref, w_ref, o_ref):
    o_ref[...] = jax.lax.dot_general(
        x_ref[...], w_ref[...], (((1,), (0,)), ((), ())),
        preferred_element_type=jnp.float32)


def _tc_matvec(x, w_col):
    blk = 2048
    return pl.pallas_call(
        _mv_body,
        grid=(B // blk,),
        in_specs=[
            pl.BlockSpec((blk, P), lambda i: (i, 0)),
            pl.BlockSpec((P, 1), lambda i: (0, 0)),
        ],
        out_specs=pl.BlockSpec((blk, 1), lambda i: (i, 0)),
        out_shape=jax.ShapeDtypeStruct((B, 1), jnp.float32),
    )(x, w_col)


def kernel(x, z, sid, W_pop, table):
    lin = _repack(table.T).reshape(TR, WPR)
    p2 = _tc_matvec(x, W_pop.reshape(P, 1))
    return _sc_partial(lin, sid, z.T, p2.reshape(B))
